# baseline (speedup 1.0000x reference)
import jax
import jax.numpy as jnp
from jax import lax
from jax.experimental import pallas as pl
from jax.experimental.pallas import tpu as pltpu
from jax.experimental.pallas import tpu_sc as plsc

NC = 2
NS = 16
NW = NC * NS
CHUNK = 128
IBATCH = 8
LANES = 16


def _mesh():
  return plsc.VectorSubcoreMesh(
      core_axis_name="c", subcore_axis_name="s", num_cores=NC,
      num_subcores=NS)


def _sc_aggregate(n_pad, e_pad, d):
  e_per_tile = e_pad // NW
  n_chunks = e_per_tile // CHUNK
  rows_per_tile = n_pad // NS

  def body(y_hbm, eil_hbm, agg_out, idx8, rows_v, agg_sh, sem):
    cid = lax.axis_index("c")
    sid = lax.axis_index("s")

    def zrow(i, _):
      def zcol(j, _):
        rows_v[i, pl.ds(j * LANES, LANES)] = jnp.zeros((LANES,), jnp.float32)
        return 0
      return lax.fori_loop(0, d // LANES, zcol, 0)
    lax.fori_loop(0, CHUNK, zrow, 0)

    r0 = sid * rows_per_tile
    n_zcopies = rows_per_tile // CHUNK
    for k in range(n_zcopies):
      pltpu.sync_copy(rows_v, agg_sh.at[pl.ds(r0 + k * CHUNK, CHUNK)])
    rem = rows_per_tile - n_zcopies * CHUNK
    if rem:
      pltpu.sync_copy(rows_v.at[pl.ds(0, rem)],
                      agg_sh.at[pl.ds(r0 + n_zcopies * CHUNK, rem)])

    plsc.subcore_barrier()

    wid = cid * NS + sid
    c0 = wid * n_chunks

    def step(g, _):
      base = pl.multiple_of(2 * (c0 + g * IBATCH), 2 * IBATCH)
      pltpu.sync_copy(eil_hbm.at[pl.ds(base, 2 * IBATCH)], idx8)
      for k in range(IBATCH):
        pltpu.async_copy(y_hbm.at[idx8.at[2 * k]], rows_v, sem).wait()
        pltpu.sync_copy(rows_v, agg_sh.at[idx8.at[2 * k + 1]], add=True)
      return 0
    lax.fori_loop(0, n_chunks // IBATCH, step, 0)

    plsc.subcore_barrier()

    pltpu.sync_copy(agg_sh.at[pl.ds(r0, rows_per_tile)],
                    agg_out.at[cid, pl.ds(r0, rows_per_tile)])

  return pl.kernel(
      body,
      out_type=[jax.ShapeDtypeStruct((NC, n_pad, d), jnp.float32)],
      mesh=_mesh(),
      scratch_types=[
          pltpu.VMEM((2 * IBATCH, CHUNK), jnp.int32),
          pltpu.VMEM((CHUNK, d), jnp.float32),
          pltpu.VMEM_SHARED((n_pad, d), jnp.float32),
          pltpu.SemaphoreType.DMA,
      ])


def _sc_degree(n_pad, e_pad, d):
  e_per_tile = e_pad // NW
  n_chunks = e_per_tile // CHUNK
  rows_per_tile = n_pad // NS

  def body(dst_hbm, deg_out, dst8, rows_v, deg_sh):
    cid = lax.axis_index("c")
    sid = lax.axis_index("s")

    def fill(val):
      def frow(i, _):
        def fcol(j, _):
          rows_v[i, pl.ds(j * LANES, LANES)] = jnp.full(
              (LANES,), val, jnp.float32)
          return 0
        return lax.fori_loop(0, d // LANES, fcol, 0)
      lax.fori_loop(0, CHUNK, frow, 0)

    fill(0.0)
    r0 = sid * rows_per_tile
    n_zcopies = rows_per_tile // CHUNK
    for k in range(n_zcopies):
      pltpu.sync_copy(rows_v, deg_sh.at[pl.ds(r0 + k * CHUNK, CHUNK)])
    rem = rows_per_tile - n_zcopies * CHUNK
    if rem:
      pltpu.sync_copy(rows_v.at[pl.ds(0, rem)],
                      deg_sh.at[pl.ds(r0 + n_zcopies * CHUNK, rem)])

    plsc.subcore_barrier()

    fill(1.0)
    wid = cid * NS + sid
    c0 = wid * n_chunks

    def step(g, _):
      base = pl.multiple_of(c0 + g * IBATCH, IBATCH)
      pltpu.sync_copy(dst_hbm.at[pl.ds(base, IBATCH)], dst8)
      for k in range(IBATCH):
        pltpu.sync_copy(rows_v, deg_sh.at[dst8.at[k]], add=True)
      return 0
    lax.fori_loop(0, n_chunks // IBATCH, step, 0)

    plsc.subcore_barrier()

    pltpu.sync_copy(deg_sh.at[pl.ds(r0, rows_per_tile)],
                    deg_out.at[cid, pl.ds(r0, rows_per_tile)])

  return pl.kernel(
      body,
      out_type=[jax.ShapeDtypeStruct((NC, n_pad, d), jnp.float32)],
      mesh=_mesh(),
      scratch_types=[
          pltpu.VMEM((IBATCH, CHUNK), jnp.int32),
          pltpu.VMEM((CHUNK, d), jnp.float32),
          pltpu.VMEM_SHARED((n_pad, d), jnp.float32),
      ])


def _dot(a, b):
  return jnp.dot(a, b, preferred_element_type=jnp.float32,
                 precision=lax.Precision.HIGHEST)


def _tc_pre(x_ref, wn1_ref, ws1_ref, b1_ref, y1_ref, s1_ref):
  x = x_ref[...]
  y1_ref[...] = _dot(x, wn1_ref[...])
  s1_ref[...] = _dot(x, ws1_ref[...]) + b1_ref[...]


def _tc_mid(s1_ref, agg_ref, deg_ref, wn2_ref, wp_ref, h_ref, y2_ref):
  agg = agg_ref[0] + agg_ref[1]
  rdeg = 1.0 / jnp.maximum(deg_ref[0] + deg_ref[1], 1.0)
  h = jnp.maximum(s1_ref[...] + agg * rdeg, 0.0)
  h_ref[...] = h
  y2_ref[...] = _dot(_dot(h, wn2_ref[...]), wp_ref[...])


def _tc_post(h_ref, agg_ref, deg_ref, ws2_ref, wp_ref, b2p_ref, out_ref):
  agg = agg_ref[0] + agg_ref[1]
  rdeg = 1.0 / jnp.maximum(deg_ref[0] + deg_ref[1], 1.0)
  out_ref[...] = (_dot(_dot(h_ref[...], ws2_ref[...]), wp_ref[...])
                  + agg * rdeg + b2p_ref[...])


def kernel(x, edge_index, W_self1, W_neigh1, b1, W_self2, W_neigh2, b2,
           W_proj, b_proj):
  n, d = x.shape
  e = edge_index.shape[1]

  n_pad = ((n + 1 + NS * 8 - 1) // (NS * 8)) * (NS * 8)
  egran = NW * CHUNK * IBATCH
  e_pad = ((e + egran - 1) // egran) * egran

  src = edge_index[0].astype(jnp.int32)
  dst = edge_index[1].astype(jnp.int32)
  pad = e_pad - e
  if pad:
    src = jnp.concatenate([src, jnp.zeros((pad,), jnp.int32)])
    dst = jnp.concatenate([dst, jnp.full((pad,), n, jnp.int32)])

  blk = 2000
  grid = (n // blk,)
  row_spec = pl.BlockSpec((blk, d), lambda i: (i, 0))
  w_spec = pl.BlockSpec((d, d), lambda i: (0, 0))
  b_spec = pl.BlockSpec((1, d), lambda i: (0, 0))
  agg_spec = pl.BlockSpec((NC, blk, d), lambda i: (0, i, 0))
  row_out = jax.ShapeDtypeStruct((n, d), jnp.float32)

  y1, s1 = pl.pallas_call(
      _tc_pre, grid=grid,
      in_specs=[row_spec, w_spec, w_spec, b_spec],
      out_specs=[row_spec, row_spec],
      out_shape=[row_out, row_out],
  )(x, W_neigh1, W_self1, b1.reshape(1, d))

  eil2 = jnp.stack([src.reshape(-1, CHUNK), dst.reshape(-1, CHUNK)],
                   axis=1).reshape(-1, CHUNK)
  dst2 = dst.reshape(-1, CHUNK)

  sc_agg = _sc_aggregate(n_pad, e_pad, d)
  sc_deg = _sc_degree(n_pad, e_pad, d)
  (deg,) = sc_deg(dst2)
  (agg1,) = sc_agg(y1, eil2)

  h, y2 = pl.pallas_call(
      _tc_mid, grid=grid,
      in_specs=[row_spec, agg_spec, agg_spec, w_spec, w_spec],
      out_specs=[row_spec, row_spec],
      out_shape=[row_out, row_out],
  )(s1, agg1, deg, W_neigh2, W_proj)

  (agg2,) = sc_agg(y2, eil2)

  b2p = (b2 @ W_proj + b_proj).reshape(1, d)

  out = pl.pallas_call(
      _tc_post, grid=grid,
      in_specs=[row_spec, agg_spec, agg_spec, w_spec, w_spec, b_spec],
      out_specs=row_spec,
      out_shape=row_out,
  )(h, agg2, deg, W_self2, W_proj, b2p)

  return out
